# baseline (speedup 1.0000x reference)
import functools

import jax
import jax.numpy as jnp
from jax import lax
from jax.experimental import pallas as pl
from jax.experimental.pallas import tpu as pltpu
from jax.experimental.pallas import tpu_sc as plsc

_NUM_FIELDS = 26
_VOCAB = 100000
_EMB_DIM = 32
_BATCH = 16384

_NC = 2
_NS = 16
_NW = _NC * _NS
_LANES = 16
_BLK = 128
_NCH = 4
_CBLK = _BATCH // (_NCH * _BLK)


@functools.partial(
    pl.kernel,
    out_type=jax.ShapeDtypeStruct(
        (_NUM_FIELDS, _EMB_DIM // 8, _BATCH // _BLK, 8, _BLK), jnp.float32
    ),
    mesh=plsc.VectorSubcoreMesh(core_axis_name="c", subcore_axis_name="s"),
    compiler_params=pltpu.CompilerParams(needs_layout_passes=False),
    scratch_types=[
        pltpu.VMEM((_VOCAB,), jnp.float32),
        pltpu.VMEM((_CBLK, _BLK), jnp.int32),
        pltpu.VMEM((_CBLK, _BLK), jnp.int32),
        pltpu.VMEM((_CBLK, _BLK), jnp.float32),
        pltpu.VMEM((_CBLK, _BLK), jnp.float32),
        pltpu.SemaphoreType.DMA,
        pltpu.SemaphoreType.DMA,
        pltpu.SemaphoreType.DMA,
        pltpu.SemaphoreType.DMA,
    ],
)
def _mce_gather(
    idx_hbm, tab_hbm, out_hbm,
    row_v, idx_a, idx_b, val_a, val_b, row_sem, idx_sem, sem_a, sem_b,
):
    w = lax.axis_index("s") * _NC + lax.axis_index("c")
    tr = w // 8
    dr = w % 8
    idx_bufs = (idx_a, idx_b)
    val_bufs = ((val_a, sem_a), (val_b, sem_b))

    def out_slice(f, c):
        return out_hbm.at[f, tr, pl.ds(c * _CBLK, _CBLK), dr]

    def field_body(f, carry):
        row_cp = pltpu.async_copy(tab_hbm.at[f, w], row_v, row_sem)
        pltpu.sync_copy(idx_hbm.at[f, 0], idx_bufs[0])
        row_cp.wait()

        for c in range(_NCH):
            ib = idx_bufs[c % 2]
            val_v, osem = val_bufs[c % 2]
            if c > 0:
                pltpu.make_async_copy(idx_hbm.at[f, c], ib, idx_sem).wait()
            if c < _NCH - 1:
                pltpu.async_copy(idx_hbm.at[f, c + 1], idx_bufs[(c + 1) % 2], idx_sem)

            if c >= 2:
                pltpu.make_async_copy(val_v, out_slice(f, c), osem).wait()
            else:

                @pl.when(f > 0)
                def _():
                    pltpu.make_async_copy(val_v, out_slice(f, c), osem).wait()

            @plsc.parallel_loop(0, _CBLK, unroll=4)
            def _(k):
                for j in range(_BLK // _LANES):
                    sl = pl.ds(j * _LANES, _LANES)
                    vals = plsc.load_gather(row_v, [ib[k, sl]])
                    val_v[k, sl] = vals

            pltpu.async_copy(val_v, out_slice(f, c), osem)
        return carry

    lax.fori_loop(0, _NUM_FIELDS, field_body, 0)
    pltpu.make_async_copy(val_a, out_slice(_NUM_FIELDS - 1, 2), sem_a).wait()
    pltpu.make_async_copy(val_b, out_slice(_NUM_FIELDS - 1, 3), sem_b).wait()


def kernel(inputs, tables):
    idx = inputs.astype(jnp.int32).T.reshape(_NUM_FIELDS, _NCH, _CBLK, _BLK)
    tt = jnp.swapaxes(tables, 1, 2)
    out5 = _mce_gather(idx, tt)
    out = out5.transpose(0, 2, 4, 1, 3).reshape(_NUM_FIELDS, _BATCH, _EMB_DIM)
    return out.reshape(_NUM_FIELDS, _BATCH, 1, _EMB_DIM)


# speedup vs baseline: 1.0224x; 1.0224x over previous
import functools

import jax
import jax.numpy as jnp
from jax import lax
from jax.experimental import pallas as pl
from jax.experimental.pallas import tpu as pltpu
from jax.experimental.pallas import tpu_sc as plsc

_NUM_FIELDS = 26
_VOCAB = 100000
_EMB_DIM = 32
_BATCH = 16384

_NC = 2
_NS = 16
_NW = _NC * _NS
_LANES = 16
_BLK = 128
_NCH = 2
_CBLK = _BATCH // (_NCH * _BLK)


@functools.partial(
    pl.kernel,
    out_type=jax.ShapeDtypeStruct(
        (_NUM_FIELDS, _EMB_DIM // 8, _BATCH // _BLK, 8, _BLK), jnp.float32
    ),
    mesh=plsc.VectorSubcoreMesh(core_axis_name="c", subcore_axis_name="s"),
    compiler_params=pltpu.CompilerParams(needs_layout_passes=False),
    scratch_types=[
        pltpu.VMEM((_VOCAB,), jnp.float32),
        pltpu.VMEM((_CBLK, _BLK), jnp.int32),
        pltpu.VMEM((_CBLK, _BLK), jnp.float32),
        pltpu.VMEM((_CBLK, _BLK), jnp.float32),
        pltpu.SemaphoreType.DMA,
        pltpu.SemaphoreType.DMA,
        pltpu.SemaphoreType.DMA,
    ],
)
def _mce_gather(
    idx_hbm, tab_hbm, out_hbm, row_v, idx_v, val_a, val_b, row_sem, sem_a, sem_b
):
    w = lax.axis_index("s") * _NC + lax.axis_index("c")
    tr = w // 8
    dr = w % 8

    def out_slice(f, c):
        return out_hbm.at[f, tr, pl.ds(c * _CBLK, _CBLK), dr]

    def field_body(f, carry):
        row_cp = pltpu.async_copy(tab_hbm.at[f, w], row_v, row_sem)
        pltpu.sync_copy(idx_hbm.at[f, 0], idx_v)
        row_cp.wait()

        for c, (val_v, osem) in enumerate(((val_a, sem_a), (val_b, sem_b))):
            if c > 0:
                pltpu.sync_copy(idx_hbm.at[f, c], idx_v)

            @pl.when(f > 0)
            def _():
                pltpu.make_async_copy(val_v, out_slice(f, c), osem).wait()

            @plsc.parallel_loop(0, _CBLK, unroll=4)
            def _(k):
                for j in range(_BLK // _LANES):
                    sl = pl.ds(j * _LANES, _LANES)
                    vals = plsc.load_gather(row_v, [idx_v[k, sl]])
                    val_v[k, sl] = vals

            pltpu.async_copy(val_v, out_slice(f, c), osem)
        return carry

    lax.fori_loop(0, _NUM_FIELDS, field_body, 0)
    pltpu.make_async_copy(val_a, out_slice(_NUM_FIELDS - 1, 0), sem_a).wait()
    pltpu.make_async_copy(val_b, out_slice(_NUM_FIELDS - 1, 1), sem_b).wait()


def kernel(inputs, tables):
    idx = inputs.astype(jnp.int32).T.reshape(_NUM_FIELDS, _NCH, _CBLK, _BLK)
    tt = jnp.swapaxes(tables, 1, 2)
    out5 = _mce_gather(idx, tt)
    out = out5.transpose(0, 2, 4, 1, 3).reshape(_NUM_FIELDS, _BATCH, _EMB_DIM)
    return out.reshape(_NUM_FIELDS, _BATCH, 1, _EMB_DIM)
